# baseline (speedup 1.0000x reference)
import functools
import math

import jax
import jax.numpy as jnp
from jax import lax
from jax.experimental import pallas as pl
from jax.experimental.pallas import tpu as pltpu
from jax.experimental.pallas import tpu_sc as plsc

_NC = 2
_NS = 16
_K = 48
_GRP = 24
_RING = 3
_G = 64


def _silu(v):
    return v * jax.nn.sigmoid(v)


def _edge_msgs(attr_p, we, be, wl, bl, E_pad, EC, C, L):
    BE = 4096

    def body(attr_ref, we_ref, be_ref, wl_ref, bl_ref, *out_refs):
        ea = jnp.dot(attr_ref[...], we_ref[...],
                     preferred_element_type=jnp.float32) + be_ref[...]
        ea = _silu(ea)
        for j in range(L):
            out_refs[j][...] = jnp.dot(
                ea, wl_ref[j], preferred_element_type=jnp.float32) + bl_ref[j]

    return pl.pallas_call(
        body,
        grid=(E_pad // BE,),
        in_specs=[
            pl.BlockSpec((BE, EC), lambda i: (i, 0)),
            pl.BlockSpec((EC, C), lambda i: (0, 0)),
            pl.BlockSpec((1, C), lambda i: (0, 0)),
            pl.BlockSpec((L, C, C), lambda i: (0, 0, 0)),
            pl.BlockSpec((L, 1, C), lambda i: (0, 0, 0)),
        ],
        out_specs=[pl.BlockSpec((BE, C), lambda i: (i, 0))] * L,
        out_shape=[jax.ShapeDtypeStruct((E_pad, C), jnp.float32)] * L,
    )(attr_p, we, be, wl, bl)


def _make_sc_agg(N, C, N_ACC, CH):
    mesh = plsc.VectorSubcoreMesh(core_axis_name="c", subcore_axis_name="s")

    @functools.partial(
        pl.kernel,
        out_type=jax.ShapeDtypeStruct((_NC, N_ACC, C), jnp.float32),
        mesh=mesh,
        scratch_types=[
            pltpu.VMEM((_GRP, _K), jnp.int32),
            pltpu.VMEM((_GRP, _K), jnp.int32),
            pltpu.VMEM((_K, C), jnp.float32),
            pltpu.VMEM((_K, C), jnp.float32),
            pltpu.VMEM((_K, C), jnp.float32),
            pltpu.VMEM((_K, C), jnp.float32),
            pltpu.VMEM((_K, C), jnp.float32),
            pltpu.VMEM((_K, C), jnp.float32),
            pltpu.VMEM((_K, C), jnp.float32),
            pltpu.VMEM_SHARED((N_ACC, C), jnp.float32),
            pltpu.SemaphoreType.DMA, pltpu.SemaphoreType.DMA,
            pltpu.SemaphoreType.DMA, pltpu.SemaphoreType.DMA,
            pltpu.SemaphoreType.DMA, pltpu.SemaphoreType.DMA,
            pltpu.SemaphoreType.DMA,
        ],
    )
    def sc_agg(x_hbm, msg_hbm, src_hbm, dst_hbm, zero_hbm, out_hbm,
               srcv, dstv, r0, r1, r2, m0, m1, m2, scat, acc,
               gsem0, gsem1, gsem2, msem0, msem1, msem2, ssem):
        rows = (r0, r1, r2)
        mrows = (m0, m1, m2)
        gsem = (gsem0, gsem1, gsem2)
        msem = (msem0, msem1, msem2)
        c = lax.axis_index("c")
        s = lax.axis_index("s")
        wid = c * _NS + s
        base = wid * CH

        def wait_kc(dst, sem):
            pltpu.make_async_copy(msg_hbm.at[pl.ds(0, _K)], dst, sem).wait()

        def issue(off, jj, b):
            pltpu.async_copy(x_hbm.at[srcv.at[jj]], rows[b], gsem[b])
            pltpu.async_copy(msg_hbm.at[pl.ds((off + jj) * _K, _K)],
                             mrows[b], msem[b])

        @pl.when(s == 0)
        def _():
            pltpu.sync_copy(zero_hbm, acc)

        plsc.subcore_barrier()

        def group(g, carry):
            off = base + g * _GRP
            pltpu.sync_copy(src_hbm.at[pl.ds(off, _GRP)], srcv)
            pltpu.sync_copy(dst_hbm.at[pl.ds(off, _GRP)], dstv)
            for jj in range(_RING):
                issue(off, jj, jj)
            for jj in range(_GRP):
                b = jj % _RING
                wait_kc(rows[b], gsem[b])
                wait_kc(mrows[b], msem[b])
                if jj >= 1:
                    wait_kc(scat, ssem)
                else:
                    @pl.when(g > 0)
                    def _():
                        wait_kc(scat, ssem)

                def rbody(r, _, b=b):
                    for cc in range(C // 16):
                        slc = pl.ds(cc * 16, 16)
                        scat[r, slc] = jnp.maximum(
                            rows[b][r, slc] + mrows[b][r, slc], 0.0)
                    return 0

                lax.fori_loop(0, _K, rbody, 0)
                pltpu.async_copy(scat, acc.at[dstv.at[jj]], ssem, add=True)
                if jj + _RING < _GRP:
                    issue(off, jj + _RING, b)
            return carry

        lax.fori_loop(0, CH // _GRP, group, 0)
        wait_kc(scat, ssem)
        plsc.subcore_barrier()

        @pl.when(s == 0)
        def _():
            pltpu.sync_copy(acc, out_hbm.at[c])

    return sc_agg


_HI = lax.Precision.HIGHEST
_DN0 = (((0,), (0,)), ((), ()))


def _onehot(b_ref):
    return (b_ref[...] == lax.broadcasted_iota(
        jnp.int32, (1, _G), 1)).astype(jnp.float32)


def _dense_layer(p, x, batch2, w1, b1, w2, b2, gw, gb, gms, N, C):
    BN = 2000
    NB = N // BN
    spec_c = lambda shp: pl.BlockSpec(shp, lambda i: tuple(0 for _ in shp))

    def d1_body(p_ref, x_ref, b_ref, w1_ref, b1_ref, w2_ref, b2_ref,
                h2_ref, mean_ref, cnt_ref, sums_ref, csc_ref):
        @pl.when(pl.program_id(0) == 0)
        def _():
            sums_ref[...] = jnp.zeros_like(sums_ref)
            csc_ref[...] = jnp.zeros_like(csc_ref)

        h = p_ref[0] + p_ref[1] + x_ref[...]
        t = jnp.dot(h, w1_ref[...], preferred_element_type=jnp.float32)
        t = _silu(t + b1_ref[...])
        h2 = jnp.dot(t, w2_ref[...],
                     preferred_element_type=jnp.float32) + b2_ref[...]
        h2_ref[...] = h2
        oh = _onehot(b_ref)
        sums_ref[...] += lax.dot_general(
            oh, h2, _DN0, preferred_element_type=jnp.float32, precision=_HI)
        csc_ref[...] += jnp.sum(oh, axis=0)[:, None]
        cnt = jnp.maximum(csc_ref[...], 1.0)
        cnt_ref[...] = cnt
        mean_ref[...] = sums_ref[...] / cnt

    h2, mean, cnt = pl.pallas_call(
        d1_body,
        grid=(NB,),
        in_specs=[
            pl.BlockSpec((_NC, BN, C), lambda i: (0, i, 0)),
            pl.BlockSpec((BN, C), lambda i: (i, 0)),
            pl.BlockSpec((BN, 1), lambda i: (i, 0)),
            spec_c((C, C)), spec_c((1, C)), spec_c((C, C)), spec_c((1, C)),
        ],
        out_specs=[pl.BlockSpec((BN, C), lambda i: (i, 0)),
                   spec_c((_G, C)), spec_c((_G, 1))],
        out_shape=[jax.ShapeDtypeStruct((N, C), jnp.float32),
                   jax.ShapeDtypeStruct((_G, C), jnp.float32),
                   jax.ShapeDtypeStruct((_G, 1), jnp.float32)],
        scratch_shapes=[pltpu.VMEM((_G, C), jnp.float32),
                        pltpu.VMEM((_G, 1), jnp.float32)],
    )(p, x, batch2, w1, b1, w2, b2)

    def d2_body(h2_ref, b_ref, mean_ref, cnt_ref, gms_ref,
                o1_ref, inv_ref, vsum_ref):
        @pl.when(pl.program_id(0) == 0)
        def _():
            vsum_ref[...] = jnp.zeros_like(vsum_ref)

        oh = _onehot(b_ref)
        o1 = h2_ref[...] - jnp.dot(
            oh, mean_ref[...], preferred_element_type=jnp.float32,
            precision=_HI) * gms_ref[...]
        o1_ref[...] = o1
        vsum_ref[...] += lax.dot_general(
            oh, o1 * o1, _DN0, preferred_element_type=jnp.float32,
            precision=_HI)
        inv_ref[...] = lax.rsqrt(vsum_ref[...] / cnt_ref[...] + 1e-5)

    o1, inv = pl.pallas_call(
        d2_body,
        grid=(NB,),
        in_specs=[
            pl.BlockSpec((BN, C), lambda i: (i, 0)),
            pl.BlockSpec((BN, 1), lambda i: (i, 0)),
            spec_c((_G, C)), spec_c((_G, 1)), spec_c((1, C)),
        ],
        out_specs=[pl.BlockSpec((BN, C), lambda i: (i, 0)),
                   spec_c((_G, C))],
        out_shape=[jax.ShapeDtypeStruct((N, C), jnp.float32),
                   jax.ShapeDtypeStruct((_G, C), jnp.float32)],
        scratch_shapes=[pltpu.VMEM((_G, C), jnp.float32)],
    )(h2, batch2, mean, cnt, gms)

    def d3_body(o1_ref, b_ref, inv_ref, gw_ref, gb_ref, out_ref):
        oh = _onehot(b_ref)
        inv_n = jnp.dot(oh, inv_ref[...], preferred_element_type=jnp.float32,
                        precision=_HI)
        out_ref[...] = gw_ref[...] * o1_ref[...] * inv_n + gb_ref[...]

    return pl.pallas_call(
        d3_body,
        grid=(NB,),
        in_specs=[
            pl.BlockSpec((BN, C), lambda i: (i, 0)),
            pl.BlockSpec((BN, 1), lambda i: (i, 0)),
            spec_c((_G, C)), spec_c((1, C)), spec_c((1, C)),
        ],
        out_specs=pl.BlockSpec((BN, C), lambda i: (i, 0)),
        out_shape=jax.ShapeDtypeStruct((N, C), jnp.float32),
    )(o1, batch2, inv, gw, gb)


def kernel(x, edge_index, edge_attr, batch, params):
    N, C = x.shape
    E = edge_index.shape[1]
    EC = edge_attr.shape[1]
    layers = params['layers']
    L = len(layers)

    NW = _NC * _NS
    q = math.lcm(_GRP, 8)
    CH = -(-((E + NW * _K - 1) // (NW * _K)) // q) * q
    E_pad = NW * _K * CH
    pad = E_pad - E
    N_ACC = N + 8

    src = edge_index[0]
    dst = edge_index[1]
    src2 = jnp.concatenate(
        [src, jnp.zeros((pad,), jnp.int32)]).reshape(NW * CH, _K)
    dst2 = jnp.concatenate(
        [dst, jnp.full((pad,), N, jnp.int32)]).reshape(NW * CH, _K)
    attr_p = jnp.concatenate(
        [edge_attr.astype(jnp.float32),
         jnp.zeros((pad, EC), jnp.float32)])
    zeros_acc = jnp.zeros((N_ACC, C), jnp.float32)
    batch2 = batch[:, None]

    wl = jnp.stack([l['Wl'] for l in layers])
    bl = jnp.stack([l['bl'] for l in layers])[:, None, :]

    msgs = _edge_msgs(attr_p, params['We'], params['be'][None, :],
                      wl, bl, E_pad, EC, C, L)

    sc_agg = _make_sc_agg(N, C, N_ACC, CH)

    xx = x
    for li, layer in enumerate(layers):
        p = sc_agg(xx, msgs[li], src2, dst2, zeros_acc)
        xx = _dense_layer(p, xx, batch2,
                          layer['W1'], layer['b1'][None, :],
                          layer['W2'], layer['b2'][None, :],
                          layer['gn_w'][None, :], layer['gn_b'][None, :],
                          layer['gn_ms'][None, :], N, C)
    return xx


# speedup vs baseline: 1.3618x; 1.3618x over previous
import functools
import math

import jax
import jax.numpy as jnp
from jax import lax
from jax.experimental import pallas as pl
from jax.experimental.pallas import tpu as pltpu
from jax.experimental.pallas import tpu_sc as plsc

_NC = 2
_NS = 16
_K = 64
_GRP = 8
_RING = 2
_C0_FRAC = 0.675
_G = 64


def _silu(v):
    return v * jax.nn.sigmoid(v)


def _edge_msgs(attr_p, we, be, wl, bl, E_pad, EC, C, L):
    BE = 4096

    def body(attr_ref, we_ref, be_ref, wl_ref, bl_ref, *out_refs):
        ea = jnp.dot(attr_ref[...], we_ref[...],
                     preferred_element_type=jnp.float32) + be_ref[...]
        ea = _silu(ea)
        for j in range(L):
            out_refs[j][...] = jnp.dot(
                ea, wl_ref[j], preferred_element_type=jnp.float32) + bl_ref[j]

    return pl.pallas_call(
        body,
        grid=(E_pad // BE,),
        in_specs=[
            pl.BlockSpec((BE, EC), lambda i: (i, 0)),
            pl.BlockSpec((EC, C), lambda i: (0, 0)),
            pl.BlockSpec((1, C), lambda i: (0, 0)),
            pl.BlockSpec((L, C, C), lambda i: (0, 0, 0)),
            pl.BlockSpec((L, 1, C), lambda i: (0, 0, 0)),
        ],
        out_specs=[pl.BlockSpec((BE, C), lambda i: (i, 0))] * L,
        out_shape=[jax.ShapeDtypeStruct((E_pad, C), jnp.float32)] * L,
    )(attr_p, we, be, wl, bl)


def _make_sc_agg(N, C, N_ACC, CH0, CH1):
    mesh = plsc.VectorSubcoreMesh(core_axis_name="c", subcore_axis_name="s")

    @functools.partial(
        pl.kernel,
        out_type=jax.ShapeDtypeStruct((_NC, N_ACC, C), jnp.float32),
        mesh=mesh,
        scratch_types=[
            pltpu.VMEM((_GRP, _K), jnp.int32),
            pltpu.VMEM((_GRP, _K), jnp.int32),
            pltpu.VMEM((_K, C), jnp.float32),
            pltpu.VMEM((_K, C), jnp.float32),
            pltpu.VMEM((_K, C), jnp.float32),
            pltpu.VMEM((_K, C), jnp.float32),
            pltpu.VMEM((_K, C), jnp.float32),
            pltpu.VMEM_SHARED((N_ACC, C), jnp.float32),
            pltpu.SemaphoreType.DMA, pltpu.SemaphoreType.DMA,
            pltpu.SemaphoreType.DMA, pltpu.SemaphoreType.DMA,
            pltpu.SemaphoreType.DMA,
        ],
    )
    def sc_agg(x_hbm, msg_hbm, src_hbm, dst_hbm, zero_hbm, out_hbm,
               srcv, dstv, r0, r1, m0, m1, scat, acc,
               gsem0, gsem1, msem0, msem1, ssem):
        rows = (r0, r1)
        mrows = (m0, m1)
        gsem = (gsem0, gsem1)
        msem = (msem0, msem1)
        c = lax.axis_index("c")
        s = lax.axis_index("s")

        def wait_kc(dst, sem):
            pltpu.make_async_copy(msg_hbm.at[pl.ds(0, _K)], dst, sem).wait()

        def issue(off, jj, b):
            pltpu.async_copy(x_hbm.at[srcv.at[jj]], rows[b], gsem[b])
            pltpu.async_copy(msg_hbm.at[pl.ds((off + jj) * _K, _K)],
                             mrows[b], msem[b])

        @pl.when(s == 0)
        def _():
            pltpu.sync_copy(zero_hbm, acc)

        plsc.subcore_barrier()

        def run_range(base, ngroups):
            def group(g, carry):
                off = base + g * _GRP
                pltpu.sync_copy(src_hbm.at[pl.ds(off, _GRP)], srcv)
                pltpu.sync_copy(dst_hbm.at[pl.ds(off, _GRP)], dstv)
                for jj in range(_RING):
                    issue(off, jj, jj)
                for jj in range(_GRP):
                    b = jj % _RING
                    wait_kc(rows[b], gsem[b])
                    wait_kc(mrows[b], msem[b])
                    if jj >= 1:
                        wait_kc(scat, ssem)
                    else:
                        @pl.when(g > 0)
                        def _():
                            wait_kc(scat, ssem)

                    def rbody(r, _, b=b):
                        for cc in range(C // 16):
                            slc = pl.ds(cc * 16, 16)
                            scat[r, slc] = jnp.maximum(
                                rows[b][r, slc] + mrows[b][r, slc], 0.0)
                        return 0

                    lax.fori_loop(0, _K, rbody, 0)
                    pltpu.async_copy(scat, acc.at[dstv.at[jj]], ssem,
                                     add=True)
                    if jj + _RING < _GRP:
                        issue(off, jj + _RING, b)
                return carry

            lax.fori_loop(0, ngroups, group, 0)
            wait_kc(scat, ssem)

        @pl.when(c == 0)
        def _():
            run_range(s * CH0, CH0 // _GRP)

        @pl.when(c == 1)
        def _():
            run_range(_NS * CH0 + s * CH1, CH1 // _GRP)

        plsc.subcore_barrier()

        @pl.when(s == 0)
        def _():
            pltpu.sync_copy(acc, out_hbm.at[c])

    return sc_agg


_HI = lax.Precision.HIGHEST
_DN0 = (((0,), (0,)), ((), ()))


def _onehot(b_ref):
    return (b_ref[...] == lax.broadcasted_iota(
        jnp.int32, (1, _G), 1)).astype(jnp.float32)


def _dense_layer(p, x, batch2, w1, b1, w2, b2, gw, gb, gms, N, C):
    BN = 2000
    NB = N // BN
    spec_c = lambda shp: pl.BlockSpec(shp, lambda i: tuple(0 for _ in shp))

    def d1_body(p_ref, x_ref, b_ref, w1_ref, b1_ref, w2_ref, b2_ref,
                h2_ref, mean_ref, cnt_ref, sums_ref, csc_ref):
        @pl.when(pl.program_id(0) == 0)
        def _():
            sums_ref[...] = jnp.zeros_like(sums_ref)
            csc_ref[...] = jnp.zeros_like(csc_ref)

        h = p_ref[0] + p_ref[1] + x_ref[...]
        t = jnp.dot(h, w1_ref[...], preferred_element_type=jnp.float32)
        t = _silu(t + b1_ref[...])
        h2 = jnp.dot(t, w2_ref[...],
                     preferred_element_type=jnp.float32) + b2_ref[...]
        h2_ref[...] = h2
        oh = _onehot(b_ref)
        sums_ref[...] += lax.dot_general(
            oh, h2, _DN0, preferred_element_type=jnp.float32, precision=_HI)
        csc_ref[...] += jnp.sum(oh, axis=0)[:, None]
        cnt = jnp.maximum(csc_ref[...], 1.0)
        cnt_ref[...] = cnt
        mean_ref[...] = sums_ref[...] / cnt

    h2, mean, cnt = pl.pallas_call(
        d1_body,
        grid=(NB,),
        in_specs=[
            pl.BlockSpec((_NC, BN, C), lambda i: (0, i, 0)),
            pl.BlockSpec((BN, C), lambda i: (i, 0)),
            pl.BlockSpec((BN, 1), lambda i: (i, 0)),
            spec_c((C, C)), spec_c((1, C)), spec_c((C, C)), spec_c((1, C)),
        ],
        out_specs=[pl.BlockSpec((BN, C), lambda i: (i, 0)),
                   spec_c((_G, C)), spec_c((_G, 1))],
        out_shape=[jax.ShapeDtypeStruct((N, C), jnp.float32),
                   jax.ShapeDtypeStruct((_G, C), jnp.float32),
                   jax.ShapeDtypeStruct((_G, 1), jnp.float32)],
        scratch_shapes=[pltpu.VMEM((_G, C), jnp.float32),
                        pltpu.VMEM((_G, 1), jnp.float32)],
    )(p, x, batch2, w1, b1, w2, b2)

    def d2_body(h2_ref, b_ref, mean_ref, cnt_ref, gms_ref,
                o1_ref, inv_ref, vsum_ref):
        @pl.when(pl.program_id(0) == 0)
        def _():
            vsum_ref[...] = jnp.zeros_like(vsum_ref)

        oh = _onehot(b_ref)
        o1 = h2_ref[...] - jnp.dot(
            oh, mean_ref[...], preferred_element_type=jnp.float32,
            precision=_HI) * gms_ref[...]
        o1_ref[...] = o1
        vsum_ref[...] += lax.dot_general(
            oh, o1 * o1, _DN0, preferred_element_type=jnp.float32,
            precision=_HI)
        inv_ref[...] = lax.rsqrt(vsum_ref[...] / cnt_ref[...] + 1e-5)

    o1, inv = pl.pallas_call(
        d2_body,
        grid=(NB,),
        in_specs=[
            pl.BlockSpec((BN, C), lambda i: (i, 0)),
            pl.BlockSpec((BN, 1), lambda i: (i, 0)),
            spec_c((_G, C)), spec_c((_G, 1)), spec_c((1, C)),
        ],
        out_specs=[pl.BlockSpec((BN, C), lambda i: (i, 0)),
                   spec_c((_G, C))],
        out_shape=[jax.ShapeDtypeStruct((N, C), jnp.float32),
                   jax.ShapeDtypeStruct((_G, C), jnp.float32)],
        scratch_shapes=[pltpu.VMEM((_G, C), jnp.float32)],
    )(h2, batch2, mean, cnt, gms)

    def d3_body(o1_ref, b_ref, inv_ref, gw_ref, gb_ref, out_ref):
        oh = _onehot(b_ref)
        inv_n = jnp.dot(oh, inv_ref[...], preferred_element_type=jnp.float32,
                        precision=_HI)
        out_ref[...] = gw_ref[...] * o1_ref[...] * inv_n + gb_ref[...]

    return pl.pallas_call(
        d3_body,
        grid=(NB,),
        in_specs=[
            pl.BlockSpec((BN, C), lambda i: (i, 0)),
            pl.BlockSpec((BN, 1), lambda i: (i, 0)),
            spec_c((_G, C)), spec_c((1, C)), spec_c((1, C)),
        ],
        out_specs=pl.BlockSpec((BN, C), lambda i: (i, 0)),
        out_shape=jax.ShapeDtypeStruct((N, C), jnp.float32),
    )(o1, batch2, inv, gw, gb)


def kernel(x, edge_index, edge_attr, batch, params):
    N, C = x.shape
    E = edge_index.shape[1]
    EC = edge_attr.shape[1]
    layers = params['layers']
    L = len(layers)

    NW = _NC * _NS
    q = math.lcm(_GRP, 8)
    CH = -(-((E + NW * _K - 1) // (NW * _K)) // q) * q
    CH0 = int(round(2 * CH * _C0_FRAC / q)) * q
    CH1 = 2 * CH - CH0
    E_pad = NW * _K * CH
    pad = E_pad - E
    N_ACC = N + 8

    src = edge_index[0]
    dst = edge_index[1]
    src2 = jnp.concatenate(
        [src, jnp.zeros((pad,), jnp.int32)]).reshape(NW * CH, _K)
    dst2 = jnp.concatenate(
        [dst, jnp.full((pad,), N, jnp.int32)]).reshape(NW * CH, _K)
    attr_p = jnp.concatenate(
        [edge_attr.astype(jnp.float32),
         jnp.zeros((pad, EC), jnp.float32)])
    zeros_acc = jnp.zeros((N_ACC, C), jnp.float32)
    batch2 = batch[:, None]

    wl = jnp.stack([l['Wl'] for l in layers])
    bl = jnp.stack([l['bl'] for l in layers])[:, None, :]

    msgs = _edge_msgs(attr_p, params['We'], params['be'][None, :],
                      wl, bl, E_pad, EC, C, L)

    sc_agg = _make_sc_agg(N, C, N_ACC, CH0, CH1)

    xx = x
    for li, layer in enumerate(layers):
        p = sc_agg(xx, msgs[li], src2, dst2, zeros_acc)
        xx = _dense_layer(p, xx, batch2,
                          layer['W1'], layer['b1'][None, :],
                          layer['W2'], layer['b2'][None, :],
                          layer['gn_w'][None, :], layer['gn_b'][None, :],
                          layer['gn_ms'][None, :], N, C)
    return xx


# speedup vs baseline: 1.4730x; 1.0817x over previous
import functools
import math

import jax
import jax.numpy as jnp
from jax import lax
from jax.experimental import pallas as pl
from jax.experimental.pallas import tpu as pltpu
from jax.experimental.pallas import tpu_sc as plsc

_NC = 2
_NS = 16
_K = 64
_GRP = 8
_RING = 2
_C0_FRAC = 0.875
_G = 64


def _silu(v):
    return v * jax.nn.sigmoid(v)


def _edge_msgs(attr_p, we, be, wl, bl, E_pad, EC, C, L):
    BE = 4096

    def body(attr_ref, we_ref, be_ref, wl_ref, bl_ref, *out_refs):
        ea = jnp.dot(attr_ref[...], we_ref[...],
                     preferred_element_type=jnp.float32) + be_ref[...]
        ea = _silu(ea)
        for j in range(L):
            out_refs[j][...] = jnp.dot(
                ea, wl_ref[j], preferred_element_type=jnp.float32) + bl_ref[j]

    return pl.pallas_call(
        body,
        grid=(E_pad // BE,),
        in_specs=[
            pl.BlockSpec((BE, EC), lambda i: (i, 0)),
            pl.BlockSpec((EC, C), lambda i: (0, 0)),
            pl.BlockSpec((1, C), lambda i: (0, 0)),
            pl.BlockSpec((L, C, C), lambda i: (0, 0, 0)),
            pl.BlockSpec((L, 1, C), lambda i: (0, 0, 0)),
        ],
        out_specs=[pl.BlockSpec((BE, C), lambda i: (i, 0))] * L,
        out_shape=[jax.ShapeDtypeStruct((E_pad, C), jnp.float32)] * L,
    )(attr_p, we, be, wl, bl)


def _make_sc_agg(N, C, N_ACC, CH0, CH1):
    mesh = plsc.VectorSubcoreMesh(core_axis_name="c", subcore_axis_name="s")

    @functools.partial(
        pl.kernel,
        out_type=jax.ShapeDtypeStruct((_NC, N_ACC, C), jnp.float32),
        mesh=mesh,
        scratch_types=[
            pltpu.VMEM((_GRP, _K), jnp.int32),
            pltpu.VMEM((_GRP, _K), jnp.int32),
            pltpu.VMEM((_K, C), jnp.float32),
            pltpu.VMEM((_K, C), jnp.float32),
            pltpu.VMEM((_K, C), jnp.float32),
            pltpu.VMEM((_K, C), jnp.float32),
            pltpu.VMEM((_K, C), jnp.float32),
            pltpu.VMEM_SHARED((N_ACC, C), jnp.float32),
            pltpu.SemaphoreType.DMA, pltpu.SemaphoreType.DMA,
            pltpu.SemaphoreType.DMA, pltpu.SemaphoreType.DMA,
            pltpu.SemaphoreType.DMA,
        ],
    )
    def sc_agg(x_hbm, msg_hbm, src_hbm, dst_hbm, zero_hbm, out_hbm,
               srcv, dstv, r0, r1, m0, m1, scat, acc,
               gsem0, gsem1, msem0, msem1, ssem):
        rows = (r0, r1)
        mrows = (m0, m1)
        gsem = (gsem0, gsem1)
        msem = (msem0, msem1)
        c = lax.axis_index("c")
        s = lax.axis_index("s")

        def wait_kc(dst, sem):
            pltpu.make_async_copy(msg_hbm.at[pl.ds(0, _K)], dst, sem).wait()

        def issue(off, jj, b):
            pltpu.async_copy(x_hbm.at[srcv.at[jj]], rows[b], gsem[b])
            pltpu.async_copy(msg_hbm.at[pl.ds((off + jj) * _K, _K)],
                             mrows[b], msem[b])

        @pl.when(s == 0)
        def _():
            pltpu.sync_copy(zero_hbm, acc)

        plsc.subcore_barrier()

        def run_range(base, ngroups):
            def group(g, carry):
                off = base + g * _GRP
                pltpu.sync_copy(src_hbm.at[pl.ds(off, _GRP)], srcv)
                pltpu.sync_copy(dst_hbm.at[pl.ds(off, _GRP)], dstv)
                for jj in range(_RING):
                    issue(off, jj, jj)
                for jj in range(_GRP):
                    b = jj % _RING
                    wait_kc(rows[b], gsem[b])
                    wait_kc(mrows[b], msem[b])
                    if jj >= 1:
                        wait_kc(scat, ssem)
                    else:
                        @pl.when(g > 0)
                        def _():
                            wait_kc(scat, ssem)

                    def rbody(r, _, b=b):
                        for cc in range(C // 16):
                            slc = pl.ds(cc * 16, 16)
                            scat[r, slc] = jnp.maximum(
                                rows[b][r, slc] + mrows[b][r, slc], 0.0)
                        return 0

                    lax.fori_loop(0, _K, rbody, 0)
                    pltpu.async_copy(scat, acc.at[dstv.at[jj]], ssem,
                                     add=True)
                    if jj + _RING < _GRP:
                        issue(off, jj + _RING, b)
                return carry

            lax.fori_loop(0, ngroups, group, 0)
            wait_kc(scat, ssem)

        @pl.when(c == 0)
        def _():
            run_range(s * CH0, CH0 // _GRP)

        @pl.when(c == 1)
        def _():
            run_range(_NS * CH0 + s * CH1, CH1 // _GRP)

        plsc.subcore_barrier()

        @pl.when(s == 0)
        def _():
            pltpu.sync_copy(acc, out_hbm.at[c])

    return sc_agg


_HI = lax.Precision.HIGHEST
_DN0 = (((0,), (0,)), ((), ()))


def _onehot(b_ref):
    return (b_ref[...] == lax.broadcasted_iota(
        jnp.int32, (1, _G), 1)).astype(jnp.float32)


def _dense_layer(p, x, batch2, w1, b1, w2, b2, gw, gb, gms, N, C):
    BN = 2000
    NB = N // BN
    spec_c = lambda shp: pl.BlockSpec(shp, lambda i: tuple(0 for _ in shp))

    def d1_body(p_ref, x_ref, b_ref, w1_ref, b1_ref, w2_ref, b2_ref,
                h2_ref, mean_ref, cnt_ref, sums_ref, csc_ref):
        @pl.when(pl.program_id(0) == 0)
        def _():
            sums_ref[...] = jnp.zeros_like(sums_ref)
            csc_ref[...] = jnp.zeros_like(csc_ref)

        h = p_ref[0] + p_ref[1] + x_ref[...]
        t = jnp.dot(h, w1_ref[...], preferred_element_type=jnp.float32)
        t = _silu(t + b1_ref[...])
        h2 = jnp.dot(t, w2_ref[...],
                     preferred_element_type=jnp.float32) + b2_ref[...]
        h2_ref[...] = h2
        oh = _onehot(b_ref)
        sums_ref[...] += lax.dot_general(
            oh, h2, _DN0, preferred_element_type=jnp.float32, precision=_HI)
        csc_ref[...] += jnp.sum(oh, axis=0)[:, None]
        cnt = jnp.maximum(csc_ref[...], 1.0)
        cnt_ref[...] = cnt
        mean_ref[...] = sums_ref[...] / cnt

    h2, mean, cnt = pl.pallas_call(
        d1_body,
        grid=(NB,),
        in_specs=[
            pl.BlockSpec((_NC, BN, C), lambda i: (0, i, 0)),
            pl.BlockSpec((BN, C), lambda i: (i, 0)),
            pl.BlockSpec((BN, 1), lambda i: (i, 0)),
            spec_c((C, C)), spec_c((1, C)), spec_c((C, C)), spec_c((1, C)),
        ],
        out_specs=[pl.BlockSpec((BN, C), lambda i: (i, 0)),
                   spec_c((_G, C)), spec_c((_G, 1))],
        out_shape=[jax.ShapeDtypeStruct((N, C), jnp.float32),
                   jax.ShapeDtypeStruct((_G, C), jnp.float32),
                   jax.ShapeDtypeStruct((_G, 1), jnp.float32)],
        scratch_shapes=[pltpu.VMEM((_G, C), jnp.float32),
                        pltpu.VMEM((_G, 1), jnp.float32)],
    )(p, x, batch2, w1, b1, w2, b2)

    def d2_body(h2_ref, b_ref, mean_ref, cnt_ref, gms_ref,
                o1_ref, inv_ref, vsum_ref):
        @pl.when(pl.program_id(0) == 0)
        def _():
            vsum_ref[...] = jnp.zeros_like(vsum_ref)

        oh = _onehot(b_ref)
        o1 = h2_ref[...] - jnp.dot(
            oh, mean_ref[...], preferred_element_type=jnp.float32,
            precision=_HI) * gms_ref[...]
        o1_ref[...] = o1
        vsum_ref[...] += lax.dot_general(
            oh, o1 * o1, _DN0, preferred_element_type=jnp.float32,
            precision=_HI)
        inv_ref[...] = lax.rsqrt(vsum_ref[...] / cnt_ref[...] + 1e-5)

    o1, inv = pl.pallas_call(
        d2_body,
        grid=(NB,),
        in_specs=[
            pl.BlockSpec((BN, C), lambda i: (i, 0)),
            pl.BlockSpec((BN, 1), lambda i: (i, 0)),
            spec_c((_G, C)), spec_c((_G, 1)), spec_c((1, C)),
        ],
        out_specs=[pl.BlockSpec((BN, C), lambda i: (i, 0)),
                   spec_c((_G, C))],
        out_shape=[jax.ShapeDtypeStruct((N, C), jnp.float32),
                   jax.ShapeDtypeStruct((_G, C), jnp.float32)],
        scratch_shapes=[pltpu.VMEM((_G, C), jnp.float32)],
    )(h2, batch2, mean, cnt, gms)

    def d3_body(o1_ref, b_ref, inv_ref, gw_ref, gb_ref, out_ref):
        oh = _onehot(b_ref)
        inv_n = jnp.dot(oh, inv_ref[...], preferred_element_type=jnp.float32,
                        precision=_HI)
        out_ref[...] = gw_ref[...] * o1_ref[...] * inv_n + gb_ref[...]

    return pl.pallas_call(
        d3_body,
        grid=(NB,),
        in_specs=[
            pl.BlockSpec((BN, C), lambda i: (i, 0)),
            pl.BlockSpec((BN, 1), lambda i: (i, 0)),
            spec_c((_G, C)), spec_c((1, C)), spec_c((1, C)),
        ],
        out_specs=pl.BlockSpec((BN, C), lambda i: (i, 0)),
        out_shape=jax.ShapeDtypeStruct((N, C), jnp.float32),
    )(o1, batch2, inv, gw, gb)


def kernel(x, edge_index, edge_attr, batch, params):
    N, C = x.shape
    E = edge_index.shape[1]
    EC = edge_attr.shape[1]
    layers = params['layers']
    L = len(layers)

    NW = _NC * _NS
    q = math.lcm(_GRP, 8)
    CH = -(-((E + NW * _K - 1) // (NW * _K)) // q) * q
    CH0 = int(round(2 * CH * _C0_FRAC / q)) * q
    CH1 = 2 * CH - CH0
    E_pad = NW * _K * CH
    pad = E_pad - E
    N_ACC = N + 8

    src = edge_index[0]
    dst = edge_index[1]
    src2 = jnp.concatenate(
        [src, jnp.zeros((pad,), jnp.int32)]).reshape(NW * CH, _K)
    dst2 = jnp.concatenate(
        [dst, jnp.full((pad,), N, jnp.int32)]).reshape(NW * CH, _K)
    attr_p = jnp.concatenate(
        [edge_attr.astype(jnp.float32),
         jnp.zeros((pad, EC), jnp.float32)])
    zeros_acc = jnp.zeros((N_ACC, C), jnp.float32)
    batch2 = batch[:, None]

    wl = jnp.stack([l['Wl'] for l in layers])
    bl = jnp.stack([l['bl'] for l in layers])[:, None, :]

    msgs = _edge_msgs(attr_p, params['We'], params['be'][None, :],
                      wl, bl, E_pad, EC, C, L)

    sc_agg = _make_sc_agg(N, C, N_ACC, CH0, CH1)

    xx = x
    for li, layer in enumerate(layers):
        p = sc_agg(xx, msgs[li], src2, dst2, zeros_acc)
        xx = _dense_layer(p, xx, batch2,
                          layer['W1'], layer['b1'][None, :],
                          layer['W2'], layer['b2'][None, :],
                          layer['gn_w'][None, :], layer['gn_b'][None, :],
                          layer['gn_ms'][None, :], N, C)
    return xx


# speedup vs baseline: 1.4949x; 1.0148x over previous
import functools
import math

import jax
import jax.numpy as jnp
from jax import lax
from jax.experimental import pallas as pl
from jax.experimental.pallas import tpu as pltpu
from jax.experimental.pallas import tpu_sc as plsc

_NC = 2
_NS = 16
_K = 64
_GRP = 8
_RING = 2
_C0_FRAC = 0.875
_G = 64


def _silu(v):
    return v * jax.nn.sigmoid(v)


def _edge_msgs(attr, we, be, wl, bl, E_pad, EC, C, L):
    E = attr.shape[0]
    BE = 4000
    assert E % BE == 0
    last = E // BE - 1
    nblk = -(-E_pad // BE)

    def body(attr_ref, we_ref, be_ref, wl_ref, bl_ref, *out_refs):
        ea = jnp.dot(attr_ref[...], we_ref[...],
                     preferred_element_type=jnp.float32) + be_ref[...]
        ea = _silu(ea)
        for j in range(L):
            out_refs[j][...] = jnp.dot(
                ea, wl_ref[j], preferred_element_type=jnp.float32) + bl_ref[j]

    return pl.pallas_call(
        body,
        grid=(nblk,),
        in_specs=[
            pl.BlockSpec((BE, EC), lambda i: (jnp.minimum(i, last), 0)),
            pl.BlockSpec((EC, C), lambda i: (0, 0)),
            pl.BlockSpec((1, C), lambda i: (0, 0)),
            pl.BlockSpec((L, C, C), lambda i: (0, 0, 0)),
            pl.BlockSpec((L, 1, C), lambda i: (0, 0, 0)),
        ],
        out_specs=[pl.BlockSpec((BE, C), lambda i: (i, 0))] * L,
        out_shape=[jax.ShapeDtypeStruct((E_pad, C), jnp.float32)] * L,
    )(attr, we, be, wl, bl)


def _make_sc_agg(N, C, N_ACC, CH0, CH1):
    mesh = plsc.VectorSubcoreMesh(core_axis_name="c", subcore_axis_name="s")

    @functools.partial(
        pl.kernel,
        out_type=jax.ShapeDtypeStruct((_NC, N_ACC, C), jnp.float32),
        mesh=mesh,
        scratch_types=[
            pltpu.VMEM((_GRP, _K), jnp.int32),
            pltpu.VMEM((_GRP, _K), jnp.int32),
            pltpu.VMEM((_K, C), jnp.float32),
            pltpu.VMEM((_K, C), jnp.float32),
            pltpu.VMEM((_K, C), jnp.float32),
            pltpu.VMEM((_K, C), jnp.float32),
            pltpu.VMEM((_K, C), jnp.float32),
            pltpu.VMEM_SHARED((N_ACC, C), jnp.float32),
            pltpu.SemaphoreType.DMA, pltpu.SemaphoreType.DMA,
            pltpu.SemaphoreType.DMA, pltpu.SemaphoreType.DMA,
            pltpu.SemaphoreType.DMA,
        ],
    )
    def sc_agg(x_hbm, msg_hbm, src_hbm, dst_hbm, zero_hbm, out_hbm,
               srcv, dstv, r0, r1, m0, m1, scat, acc,
               gsem0, gsem1, msem0, msem1, ssem):
        rows = (r0, r1)
        mrows = (m0, m1)
        gsem = (gsem0, gsem1)
        msem = (msem0, msem1)
        c = lax.axis_index("c")
        s = lax.axis_index("s")

        def wait_kc(dst, sem):
            pltpu.make_async_copy(msg_hbm.at[pl.ds(0, _K)], dst, sem).wait()

        def issue(off, jj, b):
            pltpu.async_copy(x_hbm.at[srcv.at[jj]], rows[b], gsem[b])
            pltpu.async_copy(msg_hbm.at[pl.ds((off + jj) * _K, _K)],
                             mrows[b], msem[b])

        @pl.when(s == 0)
        def _():
            pltpu.sync_copy(zero_hbm, acc)

        plsc.subcore_barrier()

        def run_range(base, ngroups):
            def group(g, carry):
                off = base + g * _GRP
                pltpu.sync_copy(src_hbm.at[pl.ds(off, _GRP)], srcv)
                pltpu.sync_copy(dst_hbm.at[pl.ds(off, _GRP)], dstv)
                for jj in range(_RING):
                    issue(off, jj, jj)
                for jj in range(_GRP):
                    b = jj % _RING
                    wait_kc(rows[b], gsem[b])
                    wait_kc(mrows[b], msem[b])
                    if jj >= 1:
                        wait_kc(scat, ssem)
                    else:
                        @pl.when(g > 0)
                        def _():
                            wait_kc(scat, ssem)

                    def rbody(r, _, b=b):
                        for cc in range(C // 16):
                            slc = pl.ds(cc * 16, 16)
                            scat[r, slc] = jnp.maximum(
                                rows[b][r, slc] + mrows[b][r, slc], 0.0)
                        return 0

                    lax.fori_loop(0, _K, rbody, 0)
                    pltpu.async_copy(scat, acc.at[dstv.at[jj]], ssem,
                                     add=True)
                    if jj + _RING < _GRP:
                        issue(off, jj + _RING, b)
                return carry

            lax.fori_loop(0, ngroups, group, 0)
            wait_kc(scat, ssem)

        @pl.when(c == 0)
        def _():
            run_range(s * CH0, CH0 // _GRP)

        @pl.when(c == 1)
        def _():
            run_range(_NS * CH0 + s * CH1, CH1 // _GRP)

        plsc.subcore_barrier()

        @pl.when(s == 0)
        def _():
            pltpu.sync_copy(acc, out_hbm.at[c])

    return sc_agg


_HI = lax.Precision.HIGHEST
_DN0 = (((0,), (0,)), ((), ()))


def _onehot(b_ref):
    return (b_ref[...] == lax.broadcasted_iota(
        jnp.int32, (1, _G), 1)).astype(jnp.float32)


def _dense_layer(p, x, batch2, w1, b1, w2, b2, gw, gb, gms, N, C):
    BN = 2000
    NB = N // BN
    spec_c = lambda shp: pl.BlockSpec(shp, lambda i: tuple(0 for _ in shp))

    def d1_body(p_ref, x_ref, b_ref, w1_ref, b1_ref, w2_ref, b2_ref,
                h2_ref, mean_ref, cnt_ref, sums_ref, csc_ref):
        @pl.when(pl.program_id(0) == 0)
        def _():
            sums_ref[...] = jnp.zeros_like(sums_ref)
            csc_ref[...] = jnp.zeros_like(csc_ref)

        h = p_ref[0] + p_ref[1] + x_ref[...]
        t = jnp.dot(h, w1_ref[...], preferred_element_type=jnp.float32)
        t = _silu(t + b1_ref[...])
        h2 = jnp.dot(t, w2_ref[...],
                     preferred_element_type=jnp.float32) + b2_ref[...]
        h2_ref[...] = h2
        oh = _onehot(b_ref)
        sums_ref[...] += lax.dot_general(
            oh, h2, _DN0, preferred_element_type=jnp.float32, precision=_HI)
        csc_ref[...] += jnp.sum(oh, axis=0)[:, None]
        cnt = jnp.maximum(csc_ref[...], 1.0)
        cnt_ref[...] = cnt
        mean_ref[...] = sums_ref[...] / cnt

    h2, mean, cnt = pl.pallas_call(
        d1_body,
        grid=(NB,),
        in_specs=[
            pl.BlockSpec((_NC, BN, C), lambda i: (0, i, 0)),
            pl.BlockSpec((BN, C), lambda i: (i, 0)),
            pl.BlockSpec((BN, 1), lambda i: (i, 0)),
            spec_c((C, C)), spec_c((1, C)), spec_c((C, C)), spec_c((1, C)),
        ],
        out_specs=[pl.BlockSpec((BN, C), lambda i: (i, 0)),
                   spec_c((_G, C)), spec_c((_G, 1))],
        out_shape=[jax.ShapeDtypeStruct((N, C), jnp.float32),
                   jax.ShapeDtypeStruct((_G, C), jnp.float32),
                   jax.ShapeDtypeStruct((_G, 1), jnp.float32)],
        scratch_shapes=[pltpu.VMEM((_G, C), jnp.float32),
                        pltpu.VMEM((_G, 1), jnp.float32)],
    )(p, x, batch2, w1, b1, w2, b2)

    def d2_body(h2_ref, b_ref, mean_ref, cnt_ref, gms_ref,
                o1_ref, inv_ref, vsum_ref):
        @pl.when(pl.program_id(0) == 0)
        def _():
            vsum_ref[...] = jnp.zeros_like(vsum_ref)

        oh = _onehot(b_ref)
        o1 = h2_ref[...] - jnp.dot(
            oh, mean_ref[...], preferred_element_type=jnp.float32,
            precision=_HI) * gms_ref[...]
        o1_ref[...] = o1
        vsum_ref[...] += lax.dot_general(
            oh, o1 * o1, _DN0, preferred_element_type=jnp.float32,
            precision=_HI)
        inv_ref[...] = lax.rsqrt(vsum_ref[...] / cnt_ref[...] + 1e-5)

    o1, inv = pl.pallas_call(
        d2_body,
        grid=(NB,),
        in_specs=[
            pl.BlockSpec((BN, C), lambda i: (i, 0)),
            pl.BlockSpec((BN, 1), lambda i: (i, 0)),
            spec_c((_G, C)), spec_c((_G, 1)), spec_c((1, C)),
        ],
        out_specs=[pl.BlockSpec((BN, C), lambda i: (i, 0)),
                   spec_c((_G, C))],
        out_shape=[jax.ShapeDtypeStruct((N, C), jnp.float32),
                   jax.ShapeDtypeStruct((_G, C), jnp.float32)],
        scratch_shapes=[pltpu.VMEM((_G, C), jnp.float32)],
    )(h2, batch2, mean, cnt, gms)

    def d3_body(o1_ref, b_ref, inv_ref, gw_ref, gb_ref, out_ref):
        oh = _onehot(b_ref)
        inv_n = jnp.dot(oh, inv_ref[...], preferred_element_type=jnp.float32,
                        precision=_HI)
        out_ref[...] = gw_ref[...] * o1_ref[...] * inv_n + gb_ref[...]

    return pl.pallas_call(
        d3_body,
        grid=(NB,),
        in_specs=[
            pl.BlockSpec((BN, C), lambda i: (i, 0)),
            pl.BlockSpec((BN, 1), lambda i: (i, 0)),
            spec_c((_G, C)), spec_c((1, C)), spec_c((1, C)),
        ],
        out_specs=pl.BlockSpec((BN, C), lambda i: (i, 0)),
        out_shape=jax.ShapeDtypeStruct((N, C), jnp.float32),
    )(o1, batch2, inv, gw, gb)


def kernel(x, edge_index, edge_attr, batch, params):
    N, C = x.shape
    E = edge_index.shape[1]
    EC = edge_attr.shape[1]
    layers = params['layers']
    L = len(layers)

    NW = _NC * _NS
    q = math.lcm(_GRP, 8)
    CH = -(-((E + NW * _K - 1) // (NW * _K)) // q) * q
    CH0 = int(round(2 * CH * _C0_FRAC / q)) * q
    CH1 = 2 * CH - CH0
    E_pad = NW * _K * CH
    pad = E_pad - E
    N_ACC = N + 8

    src = edge_index[0]
    dst = edge_index[1]
    src2 = jnp.concatenate(
        [src, jnp.zeros((pad,), jnp.int32)]).reshape(NW * CH, _K)
    dst2 = jnp.concatenate(
        [dst, jnp.full((pad,), N, jnp.int32)]).reshape(NW * CH, _K)
    zeros_acc = jnp.zeros((N_ACC, C), jnp.float32)
    batch2 = batch[:, None]

    wl = jnp.stack([l['Wl'] for l in layers])
    bl = jnp.stack([l['bl'] for l in layers])[:, None, :]

    msgs = _edge_msgs(edge_attr.astype(jnp.float32), params['We'],
                      params['be'][None, :], wl, bl, E_pad, EC, C, L)

    sc_agg = _make_sc_agg(N, C, N_ACC, CH0, CH1)

    xx = x
    for li, layer in enumerate(layers):
        p = sc_agg(xx, msgs[li], src2, dst2, zeros_acc)
        xx = _dense_layer(p, xx, batch2,
                          layer['W1'], layer['b1'][None, :],
                          layer['W2'], layer['b2'][None, :],
                          layer['gn_w'][None, :], layer['gn_b'][None, :],
                          layer['gn_ms'][None, :], N, C)
    return xx
